# baseline (speedup 1.0000x reference)
import functools

import jax
import jax.numpy as jnp
from jax import lax
from jax.experimental import pallas as pl
from jax.experimental.pallas import tpu as pltpu
from jax.experimental.pallas import tpu_sc as plsc

NC = 2
NS = 16
L = 16
NW = NC * NS

B = 16384
D = 16
BPW = B // NW
CH = 128
NCH = BPW // CH
NBLK = BPW // L

_BITREV = [int(f"{t:04b}"[::-1], 2) for t in range(L)]

_GDN = lax.GatherDimensionNumbers(
    offset_dims=(), collapsed_slice_dims=(0,), start_index_map=(0,))


def _permute(vec, perm):
    return lax.gather(vec, perm[:, None], _GDN, (1,),
                      mode=lax.GatherScatterMode.PROMISE_IN_BOUNDS)


def _sc_body(iu_hbm, ii_hbm, eu_hbm, ei_hbm, out_hbm,
             iu_v, ii_v, ru_v, ri_v, o_v, sem):
    wid = lax.axis_index("s") * NC + lax.axis_index("c")
    base = wid * BPW

    pltpu.sync_copy(iu_hbm.at[wid], iu_v)
    pltpu.sync_copy(ii_hbm.at[wid], ii_v)

    copies = []
    for j in range(NCH):
        copies.append(pltpu.async_copy(
            eu_hbm.at[iu_v.at[j]], ru_v.at[pl.ds(j * CH, CH)], sem))
        copies.append(pltpu.async_copy(
            ei_hbm.at[ii_v.at[j]], ri_v.at[pl.ds(j * CH, CH)], sem))
    for cp in copies:
        cp.wait()

    lanes = lax.iota(jnp.int32, L)
    perms = {d: lanes ^ d for d in (8, 4, 2, 1)}
    masks = {d: (lanes & d) == 0 for d in (8, 4, 2, 1)}

    def merge(a, b, d):
        s = jnp.where(masks[d], a, b)
        t = jnp.where(masks[d], b, a)
        return s + _permute(t, perms[d])

    def blk(bi, carry):
        rbase = bi * L
        p = [ru_v[rbase + _BITREV[t], :] * ri_v[rbase + _BITREV[t], :]
             for t in range(L)]
        a = [merge(p[2 * j], p[2 * j + 1], 8) for j in range(8)]
        b = [merge(a[2 * j], a[2 * j + 1], 4) for j in range(4)]
        c = [merge(b[2 * j], b[2 * j + 1], 2) for j in range(2)]
        dot = merge(c[0], c[1], 1)
        o_v[pl.ds(rbase, L)] = 1.0 / (1.0 + jnp.exp(-dot))
        return carry

    lax.fori_loop(0, NBLK, blk, 0)

    pltpu.sync_copy(o_v, out_hbm.at[pl.ds(base, BPW)])


def kernel(idx_u, idx_i, embeds_u, embeds_i):
    mesh = plsc.VectorSubcoreMesh(
        core_axis_name="c", subcore_axis_name="s",
        num_cores=NC, num_subcores=NS)

    sc = functools.partial(
        pl.kernel,
        out_type=jax.ShapeDtypeStruct((B,), jnp.float32),
        mesh=mesh,
        compiler_params=pltpu.CompilerParams(use_tc_tiling_on_sc=False),
        scratch_types=[
            pltpu.VMEM((NCH, CH), jnp.int32),
            pltpu.VMEM((NCH, CH), jnp.int32),
            pltpu.VMEM((BPW, D), jnp.float32),
            pltpu.VMEM((BPW, D), jnp.float32),
            pltpu.VMEM((BPW,), jnp.float32),
            pltpu.SemaphoreType.DMA,
        ],
    )(_sc_body)

    iu3 = idx_u.reshape(NW, NCH, CH)
    ii3 = idx_i.reshape(NW, NCH, CH)
    return sc(iu3, ii3, embeds_u, embeds_i)
